# Initial kernel scaffold; baseline (speedup 1.0000x reference)
import jax
import jax.numpy as jnp
from jax.experimental import pallas as pl


def kernel(x, edge_index, weight, bias):
    raise NotImplementedError("write your pallas kernel here")


# speedup vs baseline: 15.5180x; 15.5180x over previous
import functools

import jax
import jax.numpy as jnp
from jax import lax
from jax.experimental import pallas as pl
from jax.experimental.pallas import tpu as pltpu
from jax.experimental.pallas import tpu_sc as plsc

NC = 2
NS = 16
NW = NC * NS
CHUNK = 128


def _deg_body(dst_hbm, ones_hbm, z_hbm, cnt_hbm, idx_v, ones_v, acc_sh):
    cid = lax.axis_index("c")
    sid = lax.axis_index("s")
    wid = cid * NS + sid
    cpt = dst_hbm.shape[1]
    rpt = acc_sh.shape[0] // NS

    pltpu.sync_copy(dst_hbm.at[wid], idx_v)
    pltpu.sync_copy(ones_hbm, ones_v)
    pltpu.sync_copy(z_hbm, acc_sh.at[pl.ds(sid * rpt, rpt)])
    plsc.subcore_barrier()

    @pl.loop(0, cpt)
    def _(j):
        pltpu.sync_copy(ones_v, acc_sh.at[idx_v.at[j]], add=True)

    plsc.subcore_barrier()
    pltpu.sync_copy(acc_sh.at[pl.ds(sid * rpt, rpt)],
                    cnt_hbm.at[cid, pl.ds(sid * rpt, rpt)])


def _scatter_body(y_hbm, src_hbm, dst_hbm, z_hbm, out_hbm,
                  srcv, dstv, rows_v, acc_sh, gsem):
    cid = lax.axis_index("c")
    sid = lax.axis_index("s")
    wid = cid * NS + sid
    cpt = src_hbm.shape[1]
    rpt = acc_sh.shape[0] // NS

    pltpu.sync_copy(src_hbm.at[wid], srcv)
    pltpu.sync_copy(dst_hbm.at[wid], dstv)
    pltpu.sync_copy(z_hbm, acc_sh.at[pl.ds(sid * rpt, rpt)])
    plsc.subcore_barrier()

    @pl.loop(0, cpt)
    def _(j):
        pltpu.async_copy(y_hbm.at[srcv.at[j]], rows_v, gsem).wait()
        pltpu.sync_copy(rows_v, acc_sh.at[dstv.at[j]], add=True)

    plsc.subcore_barrier()
    pltpu.sync_copy(acc_sh.at[pl.ds(sid * rpt, rpt)],
                    out_hbm.at[cid, pl.ds(sid * rpt, rpt)])


def _y_body(x_ref, w_ref, cnt_ref, y_ref):
    deg = cnt_ref[0, :, 0:1] + cnt_ref[1, :, 0:1] + 1.0
    dis = lax.rsqrt(deg)
    y_ref[...] = dis * jnp.dot(x_ref[...], w_ref[...],
                               preferred_element_type=jnp.float32)


def _combine_body(p_ref, y_ref, cnt_ref, b_ref, o_ref):
    deg = cnt_ref[0, :, 0:1] + cnt_ref[1, :, 0:1] + 1.0
    dis = lax.rsqrt(deg)
    o_ref[...] = dis * (p_ref[0] + p_ref[1] + y_ref[...]) + b_ref[...]


def kernel(x, edge_index, weight, bias):
    n, d_in = x.shape
    d_out = weight.shape[1]
    e = edge_index.shape[1]

    src = edge_index[0].astype(jnp.int32)
    dst = edge_index[1].astype(jnp.int32)

    cpt = -(-e // (NW * CHUNK))
    e_pad = NW * cpt * CHUNK
    pad = e_pad - e
    src_p = jnp.concatenate([src, jnp.zeros((pad,), jnp.int32)])
    dst_p = jnp.concatenate([dst, jnp.full((pad,), n, jnp.int32)])
    src3 = src_p.reshape(NW, cpt, CHUNK)
    dst3 = dst_p.reshape(NW, cpt, CHUNK)

    rpt = -(-(n + 1) // (NS * 8)) * 8
    r = rpt * NS

    ones128 = jnp.ones((CHUNK, 128), jnp.float32)
    z128 = jnp.zeros((rpt, d_out), jnp.float32)

    mesh = plsc.VectorSubcoreMesh(core_axis_name="c", subcore_axis_name="s")

    deg_k = pl.kernel(
        _deg_body,
        out_type=jax.ShapeDtypeStruct((NC, r, 128), jnp.float32),
        mesh=mesh,
        scratch_types=[
            pltpu.VMEM((cpt, CHUNK), jnp.int32),
            pltpu.VMEM((CHUNK, 128), jnp.float32),
            pltpu.VMEM_SHARED((r, 128), jnp.float32),
        ],
    )
    cnt = deg_k(dst3, ones128, z128)

    rb = 400
    grid = n // rb
    y = pl.pallas_call(
        _y_body,
        grid=(grid,),
        in_specs=[
            pl.BlockSpec((rb, d_in), lambda i: (i, 0)),
            pl.BlockSpec((d_in, d_out), lambda i: (0, 0)),
            pl.BlockSpec((NC, rb, 128), lambda i: (0, i, 0)),
        ],
        out_specs=pl.BlockSpec((rb, d_out), lambda i: (i, 0)),
        out_shape=jax.ShapeDtypeStruct((n, d_out), jnp.float32),
    )(x, weight, cnt)

    scat_k = pl.kernel(
        _scatter_body,
        out_type=jax.ShapeDtypeStruct((NC, r, d_out), jnp.float32),
        mesh=mesh,
        scratch_types=[
            pltpu.VMEM((cpt, CHUNK), jnp.int32),
            pltpu.VMEM((cpt, CHUNK), jnp.int32),
            pltpu.VMEM((CHUNK, d_out), jnp.float32),
            pltpu.VMEM_SHARED((r, d_out), jnp.float32),
            pltpu.SemaphoreType.DMA,
        ],
    )
    partials = scat_k(y, src3, dst3, z128)

    out = pl.pallas_call(
        _combine_body,
        grid=(grid,),
        in_specs=[
            pl.BlockSpec((NC, rb, d_out), lambda i: (0, i, 0)),
            pl.BlockSpec((rb, d_out), lambda i: (i, 0)),
            pl.BlockSpec((NC, rb, 128), lambda i: (0, i, 0)),
            pl.BlockSpec((1, d_out), lambda i: (0, 0)),
        ],
        out_specs=pl.BlockSpec((rb, d_out), lambda i: (i, 0)),
        out_shape=jax.ShapeDtypeStruct((n, d_out), jnp.float32),
    )(partials, y, cnt, bias.reshape(1, d_out))
    return out
